# baseline (speedup 1.0000x reference)
import jax
import jax.numpy as jnp
from jax import lax
from jax.experimental import pallas as pl
from jax.experimental.pallas import tpu as pltpu
from jax.experimental.pallas import tpu_sc as plsc

B, L, N, D = 4096, 50, 4, 64
NC, NS = 2, 16
NW = NC * NS
B_PER_W = B // NW
NB = 2
IDS = NB * L * N
NCHUNK = B_PER_W // NB
GATHER_SPLITS = ((0, 128), (128, 128), (256, 128), (384, 16))
LANES = 16
NGRP = 4


def _body(table_hbm, ids_hbm, out_hbm, idx_v, gidx_v, rows_v, out_v,
          sem0, sem1, osem0, osem1):
    wid = lax.axis_index("s") * NC + lax.axis_index("c")
    wbase = wid * B_PER_W
    sems = (sem0, sem1)
    osems = (osem0, osem1)
    iota = lax.iota(jnp.int32, LANES)

    def fire(g, slot):
        b0 = wbase + g * NB
        pltpu.sync_copy(ids_hbm.at[pl.ds(b0, NB)], idx_v.at[slot])

        def build(m, carry):
            flat = m * LANES + iota
            pos = lax.shift_right_logical(flat, 2)
            b = jnp.where(pos >= L, 1, 0)
            ids16 = plsc.load_gather(idx_v.at[slot],
                                     [b, pos - b * L, flat & 3])
            gidx_v[slot, pl.ds(m * LANES, LANES)] = ids16
            return carry
        lax.fori_loop(0, IDS // LANES, build, 0)

        for off, sz in GATHER_SPLITS:
            pltpu.async_copy(table_hbm.at[gidx_v.at[slot, pl.ds(off, sz)]],
                             rows_v.at[slot, pl.ds(off, sz)], sems[slot])

    def drain(slot):
        for off, sz in GATHER_SPLITS:
            pltpu.make_async_copy(
                table_hbm.at[gidx_v.at[slot, pl.ds(off, sz)]],
                rows_v.at[slot, pl.ds(off, sz)], sems[slot]).wait()

    def process(g, slot):
        @pl.when(g >= 2)
        def _():
            b0p = wbase + (g - 2) * NB
            pltpu.make_async_copy(out_v.at[slot],
                                  out_hbm.at[pl.ds(b0p, NB)],
                                  osems[slot]).wait()

        drain(slot)

        def grp(b, k):
            off = k * LANES - jnp.where(k == NGRP - 1, 14, 0)
            rbase = (b * L + off) * N
            cnt = jnp.zeros((LANES,), jnp.int32)
            for j in range(N):
                ids_j = plsc.load_gather(gidx_v.at[slot],
                                         [rbase + iota * N + j])
                cnt = cnt + jnp.where(ids_j != 0, 1, 0)
            inv = jnp.where(
                cnt == 0, 0.0,
                jnp.where(cnt == 1, 1.0,
                          jnp.where(cnt == 2, 0.5,
                                    jnp.where(cnt == 3, 1.0 / 3.0, 0.25))))
            inv = inv.astype(jnp.float32)
            for i in range(LANES):
                invp = jnp.broadcast_to(inv[i], (LANES,))
                for d in range(D // LANES):
                    dsl = pl.ds(d * LANES, LANES)
                    acc = (rows_v[slot, rbase + 4 * i, dsl]
                           + rows_v[slot, rbase + 4 * i + 1, dsl]
                           + rows_v[slot, rbase + 4 * i + 2, dsl]
                           + rows_v[slot, rbase + 4 * i + 3, dsl])
                    out_v[slot, b, off + i, dsl] = acc * invp

        def b_loop(b, carry):
            def k_loop(k, carry2):
                grp(b, k)
                return carry2
            lax.fori_loop(0, NGRP, k_loop, 0)
            return carry
        lax.fori_loop(0, NB, b_loop, 0)

        b0 = wbase + g * NB
        pltpu.async_copy(out_v.at[slot], out_hbm.at[pl.ds(b0, NB)],
                         osems[slot])

    fire(0, 0)

    def chunk_pair(it, carry):
        for sub in range(2):
            g = 2 * it + sub

            @pl.when(g + 1 < NCHUNK)
            def _():
                fire(g + 1, 1 - sub)

            process(g, sub)
        return carry

    lax.fori_loop(0, NCHUNK // 2, chunk_pair, 0)

    for slot in range(2):
        g = NCHUNK - 2 + slot
        pltpu.make_async_copy(out_v.at[slot],
                              out_hbm.at[pl.ds(wbase + g * NB, NB)],
                              osems[slot]).wait()


@jax.jit
def kernel(token_ids, table):
    mesh = plsc.VectorSubcoreMesh(core_axis_name="c", subcore_axis_name="s")
    return pl.kernel(
        _body,
        out_type=jax.ShapeDtypeStruct((B, L, D), jnp.float32),
        mesh=mesh,
        compiler_params=pltpu.CompilerParams(use_tc_tiling_on_sc=False,
                                             needs_layout_passes=False),
        scratch_types=[
            pltpu.VMEM((2, NB, L, N), jnp.int32),
            pltpu.VMEM((2, IDS), jnp.int32),
            pltpu.VMEM((2, IDS, D), jnp.float32),
            pltpu.VMEM((2, NB, L, D), jnp.float32),
            pltpu.SemaphoreType.DMA,
            pltpu.SemaphoreType.DMA,
            pltpu.SemaphoreType.DMA,
            pltpu.SemaphoreType.DMA,
        ],
    )(table, token_ids)
